# Initial kernel scaffold; baseline (speedup 1.0000x reference)
import jax
import jax.numpy as jnp
from jax.experimental import pallas as pl


def kernel(logits, noise):
    raise NotImplementedError("write your pallas kernel here")


# speedup vs baseline: 1.3970x; 1.3970x over previous
import functools

import jax
import jax.numpy as jnp
from jax.experimental import pallas as pl
from jax.experimental.pallas import tpu as pltpu

_CHUNK = 25600


def _body(logits_ref, noise_ref, idx_ref, logp_ref, cnt_ref,
          best_score, best_idx, best_logp, cnt_acc, *, V):
    step = pl.program_id(0)
    nsteps = pl.num_programs(0)

    @pl.when(step == 0)
    def _init():
        best_score[...] = jnp.full(best_score.shape, -jnp.inf, best_score.dtype)
        best_idx[...] = jnp.zeros(best_idx.shape, best_idx.dtype)
        best_logp[...] = jnp.zeros(best_logp.shape, best_logp.dtype)
        cnt_acc[...] = jnp.zeros(cnt_acc.shape, cnt_acc.dtype)

    x = logits_ref[...]
    u = noise_ref[...]
    probs = jax.nn.sigmoid(x) * 0.999 + 0.0005
    logp = jnp.log(probs)
    gumbel = -jnp.log(-jnp.log(u))
    col = jax.lax.broadcasted_iota(jnp.int32, x.shape, 1) + step * x.shape[1]
    valid = col < V
    score = jnp.where(valid, logp + gumbel, -jnp.inf)
    gate = jnp.where(valid & (u < probs), 1.0, 0.0)

    m = jnp.max(score, axis=1, keepdims=True)
    hit = score == m
    local_idx = jnp.min(jnp.where(hit, col, V), axis=1, keepdims=True)
    logp_at = jnp.max(jnp.where(col == local_idx, logp, -jnp.inf),
                      axis=1, keepdims=True)
    local_cnt = jnp.sum(gate, axis=1, keepdims=True)

    better = m > best_score[...]
    best_idx[...] = jnp.where(better, local_idx, best_idx[...])
    best_logp[...] = jnp.where(better, logp_at, best_logp[...])
    best_score[...] = jnp.where(better, m, best_score[...])
    cnt_acc[...] = cnt_acc[...] + local_cnt

    @pl.when(step == nsteps - 1)
    def _fin():
        idx_ref[...] = best_idx[...]
        logp_ref[...] = best_logp[...]
        cnt_ref[...] = cnt_acc[...]


def kernel(logits, noise):
    B, V = logits.shape
    grid = ((V + _CHUNK - 1) // _CHUNK,)
    out_shape = [
        jax.ShapeDtypeStruct((B, 1), jnp.int32),
        jax.ShapeDtypeStruct((B, 1), jnp.float32),
        jax.ShapeDtypeStruct((B, 1), jnp.float32),
    ]
    idx, chosen_logp, cnt = pl.pallas_call(
        functools.partial(_body, V=V),
        grid=grid,
        in_specs=[
            pl.BlockSpec((B, _CHUNK), lambda i: (0, i)),
            pl.BlockSpec((B, _CHUNK), lambda i: (0, i)),
        ],
        out_specs=[
            pl.BlockSpec((B, 1), lambda i: (0, 0)),
            pl.BlockSpec((B, 1), lambda i: (0, 0)),
            pl.BlockSpec((B, 1), lambda i: (0, 0)),
        ],
        out_shape=out_shape,
        scratch_shapes=[
            pltpu.VMEM((B, 1), jnp.float32),
            pltpu.VMEM((B, 1), jnp.int32),
            pltpu.VMEM((B, 1), jnp.float32),
            pltpu.VMEM((B, 1), jnp.float32),
        ],
        compiler_params=pltpu.CompilerParams(
            dimension_semantics=("arbitrary",)),
    )(logits, noise)
    return (idx[:, 0], chosen_logp[:, 0], cnt[:, 0])


# speedup vs baseline: 1.6954x; 1.2136x over previous
import functools

import jax
import jax.numpy as jnp
from jax.experimental import pallas as pl
from jax.experimental.pallas import tpu as pltpu

_CHUNK = 25600


def _body(logits_ref, noise_ref, idx_ref, logp_ref, cnt_ref,
          best_score, best_idx, best_logp, cnt_acc, *, V):
    step = pl.program_id(0)
    nsteps = pl.num_programs(0)

    @pl.when(step == 0)
    def _init():
        best_score[...] = jnp.full(best_score.shape, -jnp.inf, best_score.dtype)
        best_idx[...] = jnp.zeros(best_idx.shape, best_idx.dtype)
        best_logp[...] = jnp.zeros(best_logp.shape, best_logp.dtype)
        cnt_acc[...] = jnp.zeros(cnt_acc.shape, cnt_acc.dtype)

    x = logits_ref[...]
    u = noise_ref[...]
    probs = jax.nn.sigmoid(x) * 0.999 + 0.0005
    w = -jnp.log(u)
    ratio = probs / w
    col = jax.lax.broadcasted_iota(jnp.int32, x.shape, 1) + step * x.shape[1]
    valid = col < V
    ratio = jnp.where(valid, ratio, -1.0)
    gate = jnp.where(valid & (u < probs), 1.0, 0.0)

    m = jnp.max(ratio, axis=1, keepdims=True)
    hit = ratio == m
    local_idx = jnp.min(jnp.where(hit, col, V), axis=1, keepdims=True)
    p_at = jnp.max(jnp.where(col == local_idx, probs, -1.0),
                   axis=1, keepdims=True)
    local_cnt = jnp.sum(gate, axis=1, keepdims=True)

    better = m > best_score[...]
    best_idx[...] = jnp.where(better, local_idx, best_idx[...])
    best_logp[...] = jnp.where(better, p_at, best_logp[...])
    best_score[...] = jnp.where(better, m, best_score[...])
    cnt_acc[...] = cnt_acc[...] + local_cnt

    @pl.when(step == nsteps - 1)
    def _fin():
        idx_ref[...] = best_idx[...]
        logp_ref[...] = jnp.log(best_logp[...])
        cnt_ref[...] = cnt_acc[...]


def kernel(logits, noise):
    B, V = logits.shape
    grid = ((V + _CHUNK - 1) // _CHUNK,)
    out_shape = [
        jax.ShapeDtypeStruct((B, 1), jnp.int32),
        jax.ShapeDtypeStruct((B, 1), jnp.float32),
        jax.ShapeDtypeStruct((B, 1), jnp.float32),
    ]
    idx, chosen_logp, cnt = pl.pallas_call(
        functools.partial(_body, V=V),
        grid=grid,
        in_specs=[
            pl.BlockSpec((B, _CHUNK), lambda i: (0, i)),
            pl.BlockSpec((B, _CHUNK), lambda i: (0, i)),
        ],
        out_specs=[
            pl.BlockSpec((B, 1), lambda i: (0, 0)),
            pl.BlockSpec((B, 1), lambda i: (0, 0)),
            pl.BlockSpec((B, 1), lambda i: (0, 0)),
        ],
        out_shape=out_shape,
        scratch_shapes=[
            pltpu.VMEM((B, 1), jnp.float32),
            pltpu.VMEM((B, 1), jnp.int32),
            pltpu.VMEM((B, 1), jnp.float32),
            pltpu.VMEM((B, 1), jnp.float32),
        ],
        compiler_params=pltpu.CompilerParams(
            dimension_semantics=("arbitrary",)),
    )(logits, noise)
    return (idx[:, 0], chosen_logp[:, 0], cnt[:, 0])
